# baseline (speedup 1.0000x reference)
import dataclasses
import math

import jax
import jax.numpy as jnp
from jax import lax
from jax.experimental import pallas as pl
from jax.experimental.pallas import tpu as pltpu
from jax.experimental.pallas import tpu_sc as plsc

_LANES = 16
_BW = 128
_SCH = 2


def _compiler_params():
    cp = pltpu.CompilerParams(use_tc_tiling_on_sc=False)
    if "needs_layout_passes" in pltpu.CompilerParams.__dataclass_fields__:
        cp = dataclasses.replace(cp, needs_layout_passes=False)
    return cp


def kernel(x, table):
    B, S = x.shape
    V, D = table.shape
    scale = float(math.sqrt(D))
    nsteps = S // _SCH

    xt = x.T

    mesh = plsc.VectorSubcoreMesh(core_axis_name="core",
                                  subcore_axis_name="subcore")

    @pl.kernel(out_type=jax.ShapeDtypeStruct((S, D, B), jnp.float32),
               mesh=mesh,
               compiler_params=_compiler_params(),
               scratch_types=[
                   pltpu.VMEM((S, _BW), jnp.int32),
                   pltpu.VMEM((2, _SCH * _BW, D), jnp.float32),
                   pltpu.VMEM((2, _SCH, D, _BW), jnp.float32),
                   pltpu.SemaphoreType.DMA,
                   pltpu.SemaphoreType.DMA,
                   pltpu.SemaphoreType.DMA,
                   pltpu.SemaphoreType.DMA,
               ])
    def emb(tbl_hbm, xt_hbm, out_hbm, idxall, gbuf, tbuf,
            gsem0, gsem1, ssem0, ssem1):
        core = lax.axis_index("core")
        sub = lax.axis_index("subcore")
        wid = sub * 2 + core
        bw = wid * _BW
        gsems = (gsem0, gsem1)
        ssems = (ssem0, ssem1)

        pltpu.sync_copy(xt_hbm.at[:, pl.ds(bw, _BW)], idxall)

        ii = lax.iota(jnp.int32, _LANES)
        rows_list = [ii + (sp * _BW + c * _LANES)
                     for sp in range(_SCH) for c in range(_BW // _LANES)]

        def start_gathers(b, g):
            for j in range(_SCH):
                s = g * _SCH + j
                pltpu.async_copy(tbl_hbm.at[idxall.at[s]],
                                 gbuf.at[b, pl.ds(j * _BW, _BW), :],
                                 gsems[b])

        def wait_gathers(b, g):
            for j in range(_SCH):
                s = g * _SCH + j
                pltpu.make_async_copy(tbl_hbm.at[idxall.at[s]],
                                      gbuf.at[b, pl.ds(j * _BW, _BW), :],
                                      gsems[b]).wait()

        def start_store(b, g):
            pltpu.async_copy(tbuf.at[b],
                             out_hbm.at[pl.ds(g * _SCH, _SCH), :,
                                        pl.ds(bw, _BW)],
                             ssems[b])

        def wait_store(b, g):
            pltpu.make_async_copy(tbuf.at[b],
                                  out_hbm.at[pl.ds(g * _SCH, _SCH), :,
                                             pl.ds(bw, _BW)],
                                  ssems[b]).wait()

        def transpose_scale(b):
            @pl.loop(0, D)
            def _(d):
                cols = jnp.zeros((_LANES,), jnp.int32) + d
                k = 0
                for sp in range(_SCH):
                    for c in range(_BW // _LANES):
                        v = plsc.load_gather(gbuf.at[b], [rows_list[k], cols])
                        tbuf.at[b, sp, d, pl.ds(c * _LANES, _LANES)][...] = (
                            v * scale)
                        k += 1

        def body(b, g, do_wait_store, do_prepare):
            wait_gathers(b, g)
            if do_wait_store:
                wait_store(b, g - 2)
            transpose_scale(b)
            start_store(b, g)
            if do_prepare:
                start_gathers(b, g + 2)

        start_gathers(0, 0)
        start_gathers(1, 1)
        body(0, 0, False, True)
        body(1, 1, False, True)

        @pl.loop(0, (nsteps - 4) // 2)
        def _(i):
            g = 2 * i + 2
            body(0, g, True, True)
            body(1, g + 1, True, True)

        body(0, nsteps - 2, True, False)
        body(1, nsteps - 1, True, False)
        wait_store(0, nsteps - 2)
        wait_store(1, nsteps - 1)

    out_t = emb(table, xt)
    return out_t.transpose(2, 0, 1)


# speedup vs baseline: 1.5811x; 1.5811x over previous
import dataclasses
import math

import jax
import jax.numpy as jnp
from jax import lax
from jax.experimental import pallas as pl
from jax.experimental.pallas import tpu as pltpu
from jax.experimental.pallas import tpu_sc as plsc

_LANES = 16
_BW = 128
_SCH = 2


def _compiler_params():
    cp = pltpu.CompilerParams(use_tc_tiling_on_sc=False)
    if "needs_layout_passes" in pltpu.CompilerParams.__dataclass_fields__:
        cp = dataclasses.replace(cp, needs_layout_passes=False)
    return cp


def kernel(x, table):
    B, S = x.shape
    V, D = table.shape
    scale = float(math.sqrt(D))
    nsteps = S // _SCH

    xt = x.T

    mesh = plsc.VectorSubcoreMesh(core_axis_name="core",
                                  subcore_axis_name="subcore")

    @pl.kernel(out_type=jax.ShapeDtypeStruct((S, D, B), jnp.float32),
               mesh=mesh,
               compiler_params=_compiler_params(),
               scratch_types=[
                   pltpu.VMEM((S, _BW), jnp.int32),
                   pltpu.VMEM((2, _SCH * _BW, D), jnp.float32),
                   pltpu.VMEM((2, _SCH, D, _BW + 1), jnp.float32),
                   pltpu.SemaphoreType.DMA,
                   pltpu.SemaphoreType.DMA,
                   pltpu.SemaphoreType.DMA,
                   pltpu.SemaphoreType.DMA,
               ])
    def emb(tbl_hbm, xt_hbm, out_hbm, idxall, gbuf, tbuf,
            gsem0, gsem1, ssem0, ssem1):
        core = lax.axis_index("core")
        sub = lax.axis_index("subcore")
        wid = sub * 2 + core
        bw = wid * _BW
        gsems = (gsem0, gsem1)
        ssems = (ssem0, ssem1)

        pltpu.sync_copy(xt_hbm.at[:, pl.ds(bw, _BW)], idxall)

        ii = lax.iota(jnp.int32, _LANES)
        rows_list = [ii + c * _LANES for c in range(D // _LANES)]

        def start_gathers(b, g):
            for j in range(_SCH):
                s = g * _SCH + j
                pltpu.async_copy(tbl_hbm.at[idxall.at[s]],
                                 gbuf.at[b, pl.ds(j * _BW, _BW), :],
                                 gsems[b])

        def wait_gathers(b, g):
            for j in range(_SCH):
                s = g * _SCH + j
                pltpu.make_async_copy(tbl_hbm.at[idxall.at[s]],
                                      gbuf.at[b, pl.ds(j * _BW, _BW), :],
                                      gsems[b]).wait()

        def start_store(b, g):
            pltpu.async_copy(tbuf.at[b, :, :, pl.ds(0, _BW)],
                             out_hbm.at[pl.ds(g * _SCH, _SCH), :,
                                        pl.ds(bw, _BW)],
                             ssems[b])

        def wait_store(b, g):
            pltpu.make_async_copy(tbuf.at[b, :, :, pl.ds(0, _BW)],
                                  out_hbm.at[pl.ds(g * _SCH, _SCH), :,
                                             pl.ds(bw, _BW)],
                                  ssems[b]).wait()

        def transpose_scale(b):
            @pl.loop(0, _SCH * _BW, step=4)
            def _(r0):
                for rr in range(4):
                    r = r0 + rr
                    sp = lax.shift_right_logical(r, 7)
                    col = lax.bitwise_and(r, _BW - 1)
                    idx_sp = jnp.zeros((_LANES,), jnp.int32) + sp
                    idx_col = jnp.zeros((_LANES,), jnp.int32) + col
                    for c in range(D // _LANES):
                        v = gbuf.at[b, r, pl.ds(c * _LANES, _LANES)][...]
                        plsc.store_scatter(tbuf.at[b],
                                           [idx_sp, rows_list[c], idx_col],
                                           v * scale)

        def body(b, g, do_wait_store, do_prepare):
            wait_gathers(b, g)
            if do_wait_store:
                wait_store(b, g - 2)
            transpose_scale(b)
            start_store(b, g)
            if do_prepare:
                start_gathers(b, g + 2)

        start_gathers(0, 0)
        start_gathers(1, 1)
        body(0, 0, False, True)
        body(1, 1, False, True)

        @pl.loop(0, (nsteps - 4) // 2)
        def _(i):
            g = 2 * i + 2
            body(0, g, True, True)
            body(1, g + 1, True, True)

        body(0, nsteps - 2, True, False)
        body(1, nsteps - 1, True, False)
        wait_store(0, nsteps - 2)
        wait_store(1, nsteps - 1)

    out_t = emb(table, xt)
    return out_t.transpose(2, 0, 1)


# speedup vs baseline: 2.2118x; 1.3989x over previous
import dataclasses
import math

import jax
import jax.numpy as jnp
from jax import lax
from jax.experimental import pallas as pl
from jax.experimental.pallas import tpu as pltpu
from jax.experimental.pallas import tpu_sc as plsc

_LANES = 16
_BW = 128
_SCH = 2


def _compiler_params():
    cp = pltpu.CompilerParams(use_tc_tiling_on_sc=False)
    if "needs_layout_passes" in pltpu.CompilerParams.__dataclass_fields__:
        cp = dataclasses.replace(cp, needs_layout_passes=False)
    return cp


def kernel(x, table):
    B, S = x.shape
    V, D = table.shape
    scale = float(math.sqrt(D))
    nsteps = S // _SCH

    xt = x.T

    mesh = plsc.VectorSubcoreMesh(core_axis_name="core",
                                  subcore_axis_name="subcore")

    @pl.kernel(out_type=jax.ShapeDtypeStruct((S, D, B), jnp.float32),
               mesh=mesh,
               compiler_params=_compiler_params(),
               scratch_types=[
                   pltpu.VMEM((S, _BW), jnp.int32),
                   pltpu.VMEM((2, _SCH * _BW, D), jnp.float32),
                   pltpu.VMEM((2, _SCH, D, _BW + 1), jnp.float32),
                   pltpu.SemaphoreType.DMA,
                   pltpu.SemaphoreType.DMA,
                   pltpu.SemaphoreType.DMA,
                   pltpu.SemaphoreType.DMA,
               ])
    def emb(tbl_hbm, xt_hbm, out_hbm, idxall, gbuf, tbuf,
            gsem0, gsem1, ssem0, ssem1):
        core = lax.axis_index("core")
        sub = lax.axis_index("subcore")
        wid = sub * 2 + core
        bw = wid * _BW
        gsems = (gsem0, gsem1)
        ssems = (ssem0, ssem1)

        pltpu.sync_copy(xt_hbm.at[:, pl.ds(bw, _BW)], idxall)

        ii = lax.iota(jnp.int32, _LANES)
        rows_list = [ii + c * _LANES for c in range(D // _LANES)]
        sp_list = [jnp.zeros((_LANES,), jnp.int32) + sp for sp in range(_SCH)]

        def start_gathers(b, g):
            for j in range(_SCH):
                s = g * _SCH + j
                pltpu.async_copy(tbl_hbm.at[idxall.at[s]],
                                 gbuf.at[b, pl.ds(j * _BW, _BW), :],
                                 gsems[b])

        def wait_gathers(b, g):
            for j in range(_SCH):
                s = g * _SCH + j
                pltpu.make_async_copy(tbl_hbm.at[idxall.at[s]],
                                      gbuf.at[b, pl.ds(j * _BW, _BW), :],
                                      gsems[b]).wait()

        def start_store(b, g):
            pltpu.async_copy(tbuf.at[b, :, :, pl.ds(0, _BW)],
                             out_hbm.at[pl.ds(g * _SCH, _SCH), :,
                                        pl.ds(bw, _BW)],
                             ssems[b])

        def wait_store(b, g):
            pltpu.make_async_copy(tbuf.at[b, :, :, pl.ds(0, _BW)],
                                  out_hbm.at[pl.ds(g * _SCH, _SCH), :,
                                             pl.ds(bw, _BW)],
                                  ssems[b]).wait()

        def transpose_scale(b):
            for sp in range(_SCH):
                @plsc.parallel_loop(0, _BW, unroll=4)
                def _(r):
                    idx_col = jnp.zeros((_LANES,), jnp.int32) + r
                    for c in range(D // _LANES):
                        v = gbuf.at[b, sp * _BW + r,
                                    pl.ds(c * _LANES, _LANES)][...]
                        plsc.store_scatter(tbuf.at[b],
                                           [sp_list[sp], rows_list[c],
                                            idx_col],
                                           v * scale)

        def body(b, g, do_wait_store, do_prepare):
            wait_gathers(b, g)
            if do_wait_store:
                wait_store(b, g - 2)
            transpose_scale(b)
            start_store(b, g)
            if do_prepare:
                start_gathers(b, g + 2)

        start_gathers(0, 0)
        start_gathers(1, 1)
        body(0, 0, False, True)
        body(1, 1, False, True)

        @pl.loop(0, (nsteps - 4) // 2)
        def _(i):
            g = 2 * i + 2
            body(0, g, True, True)
            body(1, g + 1, True, True)

        body(0, nsteps - 2, True, False)
        body(1, nsteps - 1, True, False)
        wait_store(0, nsteps - 2)
        wait_store(1, nsteps - 1)

    out_t = emb(table, xt)
    return out_t.transpose(2, 0, 1)
